# baseline (speedup 1.0000x reference)
import jax
import jax.numpy as jnp
from jax.experimental import pallas as pl
from jax.experimental.pallas import tpu as pltpu

B, S, D_MODEL = 1, 2048, 768
N_EXPERTS, TOP_K = 8, 2
D_FFN = int(D_MODEL * 2.0)
LN_EPS = 1e-5

F_BLK = 768
N_FBLK = D_FFN // F_BLK


def _moe_kernel(x_ref, rw_ref, wg_ref, wu_ref, wd_ref, g_ref, b_ref,
                out_ref):
    e = pl.program_id(0)

    xh = x_ref[...]

    logits = jax.lax.dot_general(
        xh, rw_ref[...], (((1,), (1,)), ((), ())),
        preferred_element_type=jnp.float32)
    lane = jax.lax.broadcasted_iota(jnp.int32, logits.shape, 1)
    max1 = jnp.max(logits, axis=1, keepdims=True)
    arg1 = jnp.min(jnp.where(logits == max1, lane, N_EXPERTS), axis=1,
                   keepdims=True)
    masked = jnp.where(lane == arg1, -jnp.inf, logits)
    max2 = jnp.max(masked, axis=1, keepdims=True)
    arg2 = jnp.min(jnp.where(masked == max2, lane, N_EXPERTS), axis=1,
                   keepdims=True)
    w1 = jax.nn.sigmoid(max1 - max2)
    wt = jnp.where(arg1 == e, w1, 0.0) + jnp.where(arg2 == e, 1.0 - w1, 0.0)

    eo = jnp.zeros((S, D_MODEL), jnp.float32)
    for f in range(N_FBLK):
        fs = slice(f * F_BLK, (f + 1) * F_BLK)
        gate = jax.lax.dot_general(
            xh, wg_ref[0, fs, :], (((1,), (1,)), ((), ())),
            preferred_element_type=jnp.float32).astype(jnp.bfloat16)
        up = jax.lax.dot_general(
            xh, wu_ref[0, fs, :], (((1,), (1,)), ((), ())),
            preferred_element_type=jnp.float32).astype(jnp.bfloat16)
        h = gate * jax.nn.sigmoid(gate) * up
        eo = eo + jax.lax.dot_general(
            h, wd_ref[0, :, fs], (((1,), (1,)), ((), ())),
            preferred_element_type=jnp.float32)

    contrib = wt * eo

    @pl.when(e == 0)
    def _():
        out_ref[...] = contrib

    @pl.when(e > 0)
    def _():
        out_ref[...] += contrib

    @pl.when(e == N_EXPERTS - 1)
    def _():
        o = out_ref[...]
        mean = jnp.mean(o, axis=1, keepdims=True)
        c = o - mean
        var = jnp.mean(c * c, axis=1, keepdims=True)
        out_ref[...] = c * jax.lax.rsqrt(var + LN_EPS) * g_ref[...] + b_ref[...]


def kernel(x, router_w, w_gate, w_up, w_down, ln_gamma, ln_beta):
    xh = x.reshape(S, D_MODEL).astype(jnp.bfloat16)
    rw = router_w.astype(jnp.bfloat16)
    wg = w_gate.astype(jnp.bfloat16)
    wu = w_up.astype(jnp.bfloat16)
    wd = w_down.astype(jnp.bfloat16)
    gamma = ln_gamma.reshape(1, D_MODEL)
    beta = ln_beta.reshape(1, D_MODEL)

    out = pl.pallas_call(
        _moe_kernel,
        grid=(N_EXPERTS,),
        in_specs=[
            pl.BlockSpec((S, D_MODEL), lambda e: (0, 0)),
            pl.BlockSpec((N_EXPERTS, D_MODEL), lambda e: (0, 0)),
            pl.BlockSpec((1, D_FFN, D_MODEL), lambda e: (e, 0, 0)),
            pl.BlockSpec((1, D_FFN, D_MODEL), lambda e: (e, 0, 0)),
            pl.BlockSpec((1, D_MODEL, D_FFN), lambda e: (e, 0, 0)),
            pl.BlockSpec((1, D_MODEL), lambda e: (0, 0)),
            pl.BlockSpec((1, D_MODEL), lambda e: (0, 0)),
        ],
        out_specs=pl.BlockSpec((S, D_MODEL), lambda e: (0, 0)),
        out_shape=jax.ShapeDtypeStruct((S, D_MODEL), jnp.float32),
    )(xh, rw, wg, wu, wd, gamma, beta)
    return out.reshape(B, S, D_MODEL)


# speedup vs baseline: 1.3575x; 1.3575x over previous
import jax
import jax.numpy as jnp
from jax.experimental import pallas as pl
from jax.experimental.pallas import tpu as pltpu

B, S, D_MODEL = 1, 2048, 768
N_EXPERTS, TOP_K = 8, 2
D_FFN = int(D_MODEL * 2.0)
LN_EPS = 1e-5

F_BLK = 768
N_FBLK = D_FFN // F_BLK


def _moe_kernel(x_ref, rw_ref, wg_ref, wu_ref, wd_ref, g_ref, b_ref,
                out_ref, wt_ref):
    e = pl.program_id(0)
    f = pl.program_id(1)
    first = jnp.logical_and(e == 0, f == 0)
    last = jnp.logical_and(e == N_EXPERTS - 1, f == N_FBLK - 1)

    xb = x_ref[...]

    @pl.when(first)
    def _():
        logits = jax.lax.dot_general(
            xb, rw_ref[...], (((1,), (1,)), ((), ())),
            preferred_element_type=jnp.float32)
        lane = jax.lax.broadcasted_iota(jnp.int32, logits.shape, 1)
        max1 = jnp.max(logits, axis=1, keepdims=True)
        arg1 = jnp.min(jnp.where(logits == max1, lane, N_EXPERTS), axis=1,
                       keepdims=True)
        masked = jnp.where(lane == arg1, -jnp.inf, logits)
        max2 = jnp.max(masked, axis=1, keepdims=True)
        arg2 = jnp.min(jnp.where(masked == max2, lane, N_EXPERTS), axis=1,
                       keepdims=True)
        w1 = jax.nn.sigmoid(max1 - max2)
        wt_ref[...] = (jnp.where(lane == arg1, w1, 0.0)
                       + jnp.where(lane == arg2, 1.0 - w1, 0.0))

    lane8 = jax.lax.broadcasted_iota(jnp.int32, (S, N_EXPERTS), 1)
    wt = jnp.sum(jnp.where(lane8 == e, wt_ref[...], 0.0), axis=1,
                 keepdims=True)

    gate = jax.lax.dot_general(
        xb, wg_ref[0], (((1,), (1,)), ((), ())),
        preferred_element_type=jnp.float32)
    up = jax.lax.dot_general(
        xb, wu_ref[0], (((1,), (1,)), ((), ())),
        preferred_element_type=jnp.float32)
    h = (jax.nn.silu(gate) * up).astype(jnp.bfloat16)
    eo = jax.lax.dot_general(
        h, wd_ref[0], (((1,), (1,)), ((), ())),
        preferred_element_type=jnp.float32)

    contrib = wt * eo

    @pl.when(first)
    def _():
        out_ref[...] = contrib

    @pl.when(jnp.logical_not(first))
    def _():
        out_ref[...] += contrib

    @pl.when(last)
    def _():
        o = out_ref[...]
        mean = jnp.mean(o, axis=1, keepdims=True)
        c = o - mean
        var = jnp.mean(c * c, axis=1, keepdims=True)
        out_ref[...] = c * jax.lax.rsqrt(var + LN_EPS) * g_ref[...] + b_ref[...]


def kernel(x, router_w, w_gate, w_up, w_down, ln_gamma, ln_beta):
    x2 = x.reshape(S, D_MODEL)
    gamma = ln_gamma.reshape(1, D_MODEL)
    beta = ln_beta.reshape(1, D_MODEL)

    out = pl.pallas_call(
        _moe_kernel,
        grid=(N_EXPERTS, N_FBLK),
        in_specs=[
            pl.BlockSpec((S, D_MODEL), lambda e, f: (0, 0)),
            pl.BlockSpec((N_EXPERTS, D_MODEL), lambda e, f: (0, 0)),
            pl.BlockSpec((1, F_BLK, D_MODEL), lambda e, f: (e, f, 0)),
            pl.BlockSpec((1, F_BLK, D_MODEL), lambda e, f: (e, f, 0)),
            pl.BlockSpec((1, D_MODEL, F_BLK), lambda e, f: (e, 0, f)),
            pl.BlockSpec((1, D_MODEL), lambda e, f: (0, 0)),
            pl.BlockSpec((1, D_MODEL), lambda e, f: (0, 0)),
        ],
        out_specs=pl.BlockSpec((S, D_MODEL), lambda e, f: (0, 0)),
        out_shape=jax.ShapeDtypeStruct((S, D_MODEL), jnp.float32),
        scratch_shapes=[pltpu.VMEM((S, N_EXPERTS), jnp.float32)],
    )(x2, router_w, w_gate, w_up, w_down, gamma, beta)
    return out.reshape(B, S, D_MODEL)
